# baseline (speedup 1.0000x reference)
import functools

import jax
import jax.numpy as jnp
from jax import lax
from jax.experimental import pallas as pl
from jax.experimental.pallas import tpu as pltpu
from jax.experimental.pallas import tpu_sc as plsc

_N = 10000
_E = 320000


def _exp_f32(x):
    t = x * 1.4426950408889634
    nf = (t + 12582912.0) - 12582912.0
    f = (t - nf) * 0.6931471805599453
    p = 1.0 + f * (1.0 + f * (0.5 + f * (1.0 / 6.0 + f * (
        1.0 / 24.0 + f * (1.0 / 120.0 + f * (1.0 / 720.0))))))
    n = nf.astype(jnp.int32)
    scale = jax.lax.bitcast_convert_type(
        jax.lax.shift_left(n + 127, 23), jnp.float32)
    return p * scale
_NC = 2
_NS = 16
_NW = _NC * _NS
_C = 128
_NCH = 80
_EPT = _NCH * _C
_EPAD = _NW * _EPT
_NP = 10240
_RPT = _NP // _NS
_F = 16


def _sc_edge_pass(src3, dst3, hx, as_p, ad_p):
    mesh = plsc.VectorSubcoreMesh(core_axis_name="c", subcore_axis_name="s")

    @functools.partial(
        pl.kernel,
        mesh=mesh,
        compiler_params=pltpu.CompilerParams(needs_layout_passes=False,
                                             use_tc_tiling_on_sc=False),
        out_type=[
            jax.ShapeDtypeStruct((_NC, _NP, _F), jnp.float32),
            jax.ShapeDtypeStruct((_NC, _NP), jnp.float32),
        ],
        scratch_types=[
            pltpu.VMEM((_NCH, _C), jnp.int32),
            pltpu.VMEM((_NCH, _C), jnp.int32),
            pltpu.VMEM((_NP,), jnp.float32),
            pltpu.VMEM((_NP,), jnp.float32),
            pltpu.VMEM((_C, _F), jnp.float32),
            pltpu.VMEM((_C, _F), jnp.float32),
            pltpu.VMEM((_C,), jnp.float32),
            pltpu.VMEM((_C,), jnp.float32),
            pltpu.VMEM_SHARED((_NP, _F), jnp.float32),
            pltpu.VMEM_SHARED((_NP,), jnp.float32),
            pltpu.SemaphoreType.DMA,
            pltpu.SemaphoreType.DMA,
            pltpu.SemaphoreType.DMA,
            pltpu.SemaphoreType.DMA,
            pltpu.SemaphoreType.DMA,
            pltpu.SemaphoreType.DMA,
        ],
    )
    def kern(src_h, dst_h, hx_h, as_h, ad_h, raw_h, den_h,
             src_t, dst_t, as_t, ad_t, rows0, rows1, ee0, ee1,
             raw_s, den_s, g0, g1, r0, r1, e0, e1):
        cid = lax.axis_index("c")
        sid = lax.axis_index("s")
        wid = sid * _NC + cid
        rows = (rows0, rows1)
        ees = (ee0, ee1)
        gsem = (g0, g1)
        rsem = (r0, r1)
        esem = (e0, e1)

        pltpu.sync_copy(src_h.at[wid], src_t)
        pltpu.sync_copy(dst_h.at[wid], dst_t)
        pltpu.sync_copy(as_h, as_t)
        pltpu.sync_copy(ad_h, ad_t)

        zf = jnp.zeros((_L16,), jnp.float32)

        def zrow(r, _):
            rows0[r] = zf
            return 0
        lax.fori_loop(0, _C, zrow, 0, unroll=8)

        for k in range(_C // 16):
            ee0[pl.ds(k * 16, 16)] = zf

        base = sid * _RPT
        for t in range(_RPT // _C):
            pltpu.sync_copy(rows0, raw_s.at[pl.ds(base + t * _C, _C)])
            pltpu.sync_copy(ee0, den_s.at[pl.ds(base + t * _C, _C)])
        plsc.subcore_barrier()

        pltpu.async_copy(hx_h.at[src_t.at[0]], rows0, g0)

        def j2_body(j2, _):
            for b in range(2):
                j = j2 * 2 + b
                ob = 1 - b
                rt, et = rows[b], ees[b]

                for k in range(_C // 16):
                    sidx = src_t[j, pl.ds(k * 16, 16)]
                    didx = dst_t[j, pl.ds(k * 16, 16)]
                    e = plsc.load_gather(as_t, [sidx]) + plsc.load_gather(
                        ad_t, [didx])
                    e = jnp.where(e > 0.0, e, 0.2 * e)
                    et[pl.ds(k * 16, 16)] = _exp_f32(e)

                pltpu.make_async_copy(hx_h.at[src_t.at[j]], rt, gsem[b]).wait()

                def scale_body(r, _):
                    eev = plsc.load_gather(
                        et, [jnp.full((16,), r, jnp.int32)])
                    rt[r] = rt[r] * eev
                    return 0
                lax.fori_loop(0, _C, scale_body, 0, unroll=16)

                @pl.when(j >= 1)
                def _():
                    pltpu.make_async_copy(
                        rows[ob], raw_s.at[dst_t.at[j - 1]], rsem[ob]).wait()
                    pltpu.make_async_copy(
                        ees[ob], den_s.at[dst_t.at[j - 1]], esem[ob]).wait()

                @pl.when(j + 1 < _NCH)
                def _():
                    pltpu.async_copy(
                        hx_h.at[src_t.at[j + 1]], rows[ob], gsem[ob])

                pltpu.async_copy(rt, raw_s.at[dst_t.at[j]], rsem[b], add=True)
                pltpu.async_copy(et, den_s.at[dst_t.at[j]], esem[b], add=True)
            return 0

        lax.fori_loop(0, _NCH // 2, j2_body, 0)
        pltpu.make_async_copy(
            rows1, raw_s.at[dst_t.at[_NCH - 1]], r1).wait()
        pltpu.make_async_copy(
            ee1, den_s.at[dst_t.at[_NCH - 1]], e1).wait()
        plsc.subcore_barrier()

        pltpu.sync_copy(raw_s.at[pl.ds(base, _RPT)],
                        raw_h.at[cid, pl.ds(base, _RPT)])
        pltpu.sync_copy(den_s.at[pl.ds(base, _RPT)],
                        den_h.at[cid, pl.ds(base, _RPT)])

    return kern(src3, dst3, hx, as_p, ad_p)


_L16 = 16


def _dense_first(x, W1, a1s, a1d):
    def body(x_ref, w_ref, as_ref, ad_ref, hx_ref, asp_ref, adp_ref):
        h = jnp.dot(x_ref[...], w_ref[...], preferred_element_type=jnp.float32)
        hx_ref[...] = jnp.zeros((_NP, _F), jnp.float32)
        hx_ref[:_N, :] = h
        asp_ref[...] = jnp.zeros((_NP,), jnp.float32)
        adp_ref[...] = jnp.zeros((_NP,), jnp.float32)
        asp_ref[:_N] = h @ as_ref[...]
        adp_ref[:_N] = h @ ad_ref[...]

    return pl.pallas_call(
        body,
        out_shape=[
            jax.ShapeDtypeStruct((_NP, _F), jnp.float32),
            jax.ShapeDtypeStruct((_NP,), jnp.float32),
            jax.ShapeDtypeStruct((_NP,), jnp.float32),
        ],
    )(x, W1, a1s, a1d)


def _dense_mid(raw, den, b, W, a_s, a_d, fin):
    fout = W.shape[1]

    def body(raw_ref, den_ref, b_ref, w_ref, as_ref, ad_ref,
             hx_ref, asp_ref, adp_ref):
        rawv = raw_ref[0, :_N, :fin] + raw_ref[1, :_N, :fin]
        denv = den_ref[0, :_N] + den_ref[1, :_N]
        z = rawv / (denv[:, None] + 1e-16) + b_ref[...]
        z = jnp.maximum(z, 0.0)
        h = jnp.dot(z, w_ref[...], preferred_element_type=jnp.float32)
        hx_ref[...] = jnp.zeros((_NP, _F), jnp.float32)
        hx_ref[:_N, :fout] = h
        asp_ref[...] = jnp.zeros((_NP,), jnp.float32)
        adp_ref[...] = jnp.zeros((_NP,), jnp.float32)
        asp_ref[:_N] = h @ as_ref[...]
        adp_ref[:_N] = h @ ad_ref[...]

    return pl.pallas_call(
        body,
        out_shape=[
            jax.ShapeDtypeStruct((_NP, _F), jnp.float32),
            jax.ShapeDtypeStruct((_NP,), jnp.float32),
            jax.ShapeDtypeStruct((_NP,), jnp.float32),
        ],
    )(raw, den, b, W, a_s, a_d)


def _dense_last(raw, den, b, Wfc, bfc):
    def body(raw_ref, den_ref, b_ref, w_ref, bfc_ref, o_ref):
        rawv = raw_ref[0, :_N, :8] + raw_ref[1, :_N, :8]
        denv = den_ref[0, :_N] + den_ref[1, :_N]
        z = rawv / (denv[:, None] + 1e-16) + b_ref[...]
        z = jnp.maximum(z, 0.0)
        o_ref[...] = jnp.dot(z, w_ref[...],
                             preferred_element_type=jnp.float32) + bfc_ref[...]

    return pl.pallas_call(
        body,
        out_shape=jax.ShapeDtypeStruct((_N, 1), jnp.float32),
    )(raw, den, b, Wfc, bfc[None, :])


def kernel(x, edge_index, W1, a1s, a1d, b1, W2, a2s, a2d, b2, W3, a3s, a3d, b3, Wfc, bfc):
    src = edge_index[:, 0]
    dst = edge_index[:, 1]
    padn = jnp.full((_EPAD - _E,), _N, jnp.int32)
    src3 = jnp.concatenate([src, padn]).reshape(_NW, _NCH, _C)
    dst3 = jnp.concatenate([dst, padn]).reshape(_NW, _NCH, _C)

    hx, asp, adp = _dense_first(x, W1, a1s, a1d)
    raw, den = _sc_edge_pass(src3, dst3, hx, asp, adp)
    hx, asp, adp = _dense_mid(raw, den, b1, W2, a2s, a2d, _F)
    raw, den = _sc_edge_pass(src3, dst3, hx, asp, adp)
    hx, asp, adp = _dense_mid(raw, den, b2, W3, a3s, a3d, 8)
    raw, den = _sc_edge_pass(src3, dst3, hx, asp, adp)
    return _dense_last(raw, den, b3, Wfc, bfc)


# speedup vs baseline: 1.0980x; 1.0980x over previous
import functools

import jax
import jax.numpy as jnp
from jax import lax
from jax.experimental import pallas as pl
from jax.experimental.pallas import tpu as pltpu
from jax.experimental.pallas import tpu_sc as plsc

_N = 10000
_E = 320000


def _exp_f32(x):
    t = x * 1.4426950408889634
    nf = (t + 12582912.0) - 12582912.0
    f = (t - nf) * 0.6931471805599453
    p = 1.0 + f * (1.0 + f * (0.5 + f * (1.0 / 6.0 + f * (
        1.0 / 24.0 + f * (1.0 / 120.0 + f * (1.0 / 720.0))))))
    n = nf.astype(jnp.int32)
    scale = jax.lax.bitcast_convert_type(
        jax.lax.shift_left(n + 127, 23), jnp.float32)
    return p * scale
_NC = 2
_NS = 16
_NW = _NC * _NS
_C = 128
_NCH = 80
_EPT = _NCH * _C
_EPAD = _NW * _EPT
_NP = 10240
_RPT = _NP // _NS
_F = 16


def _sc_edge_pass(src3, dst3, hx, as_p, ad_p, fold_den):
    mesh = plsc.VectorSubcoreMesh(core_axis_name="c", subcore_axis_name="s")

    @functools.partial(
        pl.kernel,
        mesh=mesh,
        compiler_params=pltpu.CompilerParams(needs_layout_passes=False,
                                             use_tc_tiling_on_sc=False),
        out_type=[
            jax.ShapeDtypeStruct((_NC, _NP, _F), jnp.float32),
            jax.ShapeDtypeStruct((_NC, _NP), jnp.float32),
        ],
        scratch_types=[
            pltpu.VMEM((_NCH, _C), jnp.int32),
            pltpu.VMEM((_NCH, _C), jnp.int32),
            pltpu.VMEM((_NP,), jnp.float32),
            pltpu.VMEM((_NP,), jnp.float32),
            pltpu.VMEM((_C, _F), jnp.float32),
            pltpu.VMEM((_C, _F), jnp.float32),
            pltpu.VMEM((_C,), jnp.float32),
            pltpu.VMEM((_C,), jnp.float32),
            pltpu.VMEM_SHARED((_NP, _F), jnp.float32),
            pltpu.VMEM_SHARED((_NP,), jnp.float32),
            pltpu.SemaphoreType.DMA,
            pltpu.SemaphoreType.DMA,
            pltpu.SemaphoreType.DMA,
            pltpu.SemaphoreType.DMA,
            pltpu.SemaphoreType.DMA,
            pltpu.SemaphoreType.DMA,
        ],
    )
    def kern(src_h, dst_h, hx_h, as_h, ad_h, raw_h, den_h,
             src_t, dst_t, as_t, ad_t, rows0, rows1, ee0, ee1,
             raw_s, den_s, g0, g1, r0, r1, e0, e1):
        cid = lax.axis_index("c")
        sid = lax.axis_index("s")
        wid = sid * _NC + cid
        rows = (rows0, rows1)
        ees = (ee0, ee1)
        gsem = (g0, g1)
        rsem = (r0, r1)
        esem = (e0, e1)

        pltpu.sync_copy(src_h.at[wid], src_t)
        pltpu.sync_copy(dst_h.at[wid], dst_t)
        pltpu.sync_copy(as_h, as_t)
        pltpu.sync_copy(ad_h, ad_t)

        zf = jnp.zeros((_L16,), jnp.float32)

        def zrow(r, _):
            rows0[r] = zf
            return 0
        lax.fori_loop(0, _C, zrow, 0, unroll=8)

        for k in range(_C // 16):
            ee0[pl.ds(k * 16, 16)] = zf

        base = sid * _RPT
        for t in range(_RPT // _C):
            pltpu.sync_copy(rows0, raw_s.at[pl.ds(base + t * _C, _C)])
            if not fold_den:
                pltpu.sync_copy(ee0, den_s.at[pl.ds(base + t * _C, _C)])
        plsc.subcore_barrier()

        pltpu.async_copy(hx_h.at[src_t.at[0]], rows0, g0)

        def j2_body(j2, _):
            for b in range(2):
                j = j2 * 2 + b
                ob = 1 - b
                rt, et = rows[b], ees[b]

                for k in range(_C // 16):
                    sidx = src_t[j, pl.ds(k * 16, 16)]
                    didx = dst_t[j, pl.ds(k * 16, 16)]
                    e = plsc.load_gather(as_t, [sidx]) + plsc.load_gather(
                        ad_t, [didx])
                    e = jnp.where(e > 0.0, e, 0.2 * e)
                    et[pl.ds(k * 16, 16)] = _exp_f32(e)

                pltpu.make_async_copy(hx_h.at[src_t.at[j]], rt, gsem[b]).wait()

                def scale_body(r, _):
                    eev = plsc.load_gather(
                        et, [jnp.full((16,), r, jnp.int32)])
                    rt[r] = rt[r] * eev
                    return 0
                lax.fori_loop(0, _C, scale_body, 0, unroll=16)

                @pl.when(j >= 1)
                def _():
                    pltpu.make_async_copy(
                        rows[ob], raw_s.at[dst_t.at[j - 1]], rsem[ob]).wait()
                    if not fold_den:
                        pltpu.make_async_copy(
                            ees[ob], den_s.at[dst_t.at[j - 1]],
                            esem[ob]).wait()

                @pl.when(j + 1 < _NCH)
                def _():
                    pltpu.async_copy(
                        hx_h.at[src_t.at[j + 1]], rows[ob], gsem[ob])

                pltpu.async_copy(rt, raw_s.at[dst_t.at[j]], rsem[b], add=True)
                if not fold_den:
                    pltpu.async_copy(
                        et, den_s.at[dst_t.at[j]], esem[b], add=True)
            return 0

        lax.fori_loop(0, _NCH // 2, j2_body, 0)
        pltpu.make_async_copy(
            rows1, raw_s.at[dst_t.at[_NCH - 1]], r1).wait()
        if not fold_den:
            pltpu.make_async_copy(
                ee1, den_s.at[dst_t.at[_NCH - 1]], e1).wait()
        plsc.subcore_barrier()

        pltpu.sync_copy(raw_s.at[pl.ds(base, _RPT)],
                        raw_h.at[cid, pl.ds(base, _RPT)])
        pltpu.sync_copy(den_s.at[pl.ds(base, _RPT)],
                        den_h.at[cid, pl.ds(base, _RPT)])

    return kern(src3, dst3, hx, as_p, ad_p)


_L16 = 16


def _dense_first(x, W1, a1s, a1d):
    def body(x_ref, w_ref, aa_ref, hx_ref, asp_ref, adp_ref):
        h = jnp.dot(x_ref[...], w_ref[...], preferred_element_type=jnp.float32)
        hx_ref[...] = jnp.zeros((_NP, _F), jnp.float32)
        hx_ref[:_N, :] = h
        pair = jnp.dot(h, aa_ref[...], preferred_element_type=jnp.float32)
        asp_ref[...] = jnp.zeros((_NP,), jnp.float32)
        adp_ref[...] = jnp.zeros((_NP,), jnp.float32)
        asp_ref[:_N] = pair[:, 0]
        adp_ref[:_N] = pair[:, 1]

    return pl.pallas_call(
        body,
        out_shape=[
            jax.ShapeDtypeStruct((_NP, _F), jnp.float32),
            jax.ShapeDtypeStruct((_NP,), jnp.float32),
            jax.ShapeDtypeStruct((_NP,), jnp.float32),
        ],
    )(x, W1, jnp.stack([a1s, a1d], axis=1))


def _dense_mid(raw, den, b, W, a_s, a_d, fin):
    fout = W.shape[1]

    def body(raw_ref, den_ref, b_ref, w_ref, aa_ref,
             hx_ref, asp_ref, adp_ref):
        rawv = raw_ref[0, :_N, :fin] + raw_ref[1, :_N, :fin]
        if fin == _F:
            denv = den_ref[0, :_N] + den_ref[1, :_N]
        else:
            denv = raw_ref[0, :_N, 8] + raw_ref[1, :_N, 8]
        z = rawv / (denv[:, None] + 1e-16) + b_ref[...]
        z = jnp.maximum(z, 0.0)
        h = jnp.dot(z, w_ref[...], preferred_element_type=jnp.float32)
        hx_ref[...] = jnp.zeros((_NP, _F), jnp.float32)
        hx_ref[:_N, :fout] = h
        hx_ref[:_N, 8] = jnp.ones((_N,), jnp.float32)
        pair = jnp.dot(h, aa_ref[...], preferred_element_type=jnp.float32)
        asp_ref[...] = jnp.zeros((_NP,), jnp.float32)
        adp_ref[...] = jnp.zeros((_NP,), jnp.float32)
        asp_ref[:_N] = pair[:, 0]
        adp_ref[:_N] = pair[:, 1]

    return pl.pallas_call(
        body,
        out_shape=[
            jax.ShapeDtypeStruct((_NP, _F), jnp.float32),
            jax.ShapeDtypeStruct((_NP,), jnp.float32),
            jax.ShapeDtypeStruct((_NP,), jnp.float32),
        ],
    )(raw, den, b, W, jnp.stack([a_s, a_d], axis=1))


def _dense_last(raw, den, b, Wfc, bfc):
    def body(raw_ref, den_ref, b_ref, w_ref, bfc_ref, o_ref):
        rawv = raw_ref[0, :_N, :8] + raw_ref[1, :_N, :8]
        denv = raw_ref[0, :_N, 8] + raw_ref[1, :_N, 8]
        z = rawv / (denv[:, None] + 1e-16) + b_ref[...]
        z = jnp.maximum(z, 0.0)
        o_ref[...] = jnp.dot(z, w_ref[...],
                             preferred_element_type=jnp.float32) + bfc_ref[...]

    return pl.pallas_call(
        body,
        out_shape=jax.ShapeDtypeStruct((_N, 1), jnp.float32),
    )(raw, den, b, Wfc, bfc[None, :])


def kernel(x, edge_index, W1, a1s, a1d, b1, W2, a2s, a2d, b2, W3, a3s, a3d, b3, Wfc, bfc):
    src = edge_index[:, 0]
    dst = edge_index[:, 1]
    padn = jnp.full((_EPAD - _E,), _N, jnp.int32)
    src3 = jnp.concatenate([src, padn]).reshape(_NW, _NCH, _C)
    dst3 = jnp.concatenate([dst, padn]).reshape(_NW, _NCH, _C)

    hx, asp, adp = _dense_first(x, W1, a1s, a1d)
    raw, den = _sc_edge_pass(src3, dst3, hx, asp, adp, False)
    hx, asp, adp = _dense_mid(raw, den, b1, W2, a2s, a2d, _F)
    raw, den = _sc_edge_pass(src3, dst3, hx, asp, adp, True)
    hx, asp, adp = _dense_mid(raw, den, b2, W3, a3s, a3d, 8)
    raw, den = _sc_edge_pass(src3, dst3, hx, asp, adp, True)
    return _dense_last(raw, den, b3, Wfc, bfc)


# speedup vs baseline: 1.2404x; 1.1297x over previous
import functools

import jax
import jax.numpy as jnp
from jax import lax
from jax.experimental import pallas as pl
from jax.experimental.pallas import tpu as pltpu
from jax.experimental.pallas import tpu_sc as plsc

_N = 10000
_E = 320000


def _exp_f32(x):
    t = x * 1.4426950408889634
    nf = (t + 12582912.0) - 12582912.0
    f = (t - nf) * 0.6931471805599453
    p = 1.0 + f * (1.0 + f * (0.5 + f * (1.0 / 6.0 + f * (
        1.0 / 24.0 + f * (1.0 / 120.0 + f * (1.0 / 720.0))))))
    n = nf.astype(jnp.int32)
    scale = jax.lax.bitcast_convert_type(
        jax.lax.shift_left(n + 127, 23), jnp.float32)
    return p * scale
_NC = 2
_NS = 16
_NW = _NC * _NS
_C = 512
_NCH = 20
_EPT = _NCH * _C
_EPAD = _NW * _EPT
_NP = 10240
_RPT = _NP // _NS
_F = 16


def _sc_edge_pass(src3, dst3, hx, as_p, ad_p, fold_den):
    mesh = plsc.VectorSubcoreMesh(core_axis_name="c", subcore_axis_name="s")

    @functools.partial(
        pl.kernel,
        mesh=mesh,
        compiler_params=pltpu.CompilerParams(needs_layout_passes=False,
                                             use_tc_tiling_on_sc=False),
        out_type=[
            jax.ShapeDtypeStruct((_NC, _NP, _F), jnp.float32),
            jax.ShapeDtypeStruct((_NC, _NP), jnp.float32),
        ],
        scratch_types=[
            pltpu.VMEM((_NCH, _C), jnp.int32),
            pltpu.VMEM((_NCH, _C), jnp.int32),
            pltpu.VMEM((_NP,), jnp.float32),
            pltpu.VMEM((_NP,), jnp.float32),
            pltpu.VMEM((_C, _F), jnp.float32),
            pltpu.VMEM((_C, _F), jnp.float32),
            pltpu.VMEM((_C,), jnp.float32),
            pltpu.VMEM((_C,), jnp.float32),
            pltpu.VMEM_SHARED((_NP, _F), jnp.float32),
            pltpu.VMEM_SHARED((_NP,), jnp.float32),
            pltpu.SemaphoreType.DMA,
            pltpu.SemaphoreType.DMA,
            pltpu.SemaphoreType.DMA,
            pltpu.SemaphoreType.DMA,
            pltpu.SemaphoreType.DMA,
            pltpu.SemaphoreType.DMA,
        ],
    )
    def kern(src_h, dst_h, hx_h, as_h, ad_h, raw_h, den_h,
             src_t, dst_t, as_t, ad_t, rows0, rows1, ee0, ee1,
             raw_s, den_s, g0, g1, r0, r1, e0, e1):
        cid = lax.axis_index("c")
        sid = lax.axis_index("s")
        wid = sid * _NC + cid
        rows = (rows0, rows1)
        ees = (ee0, ee1)
        gsem = (g0, g1)
        rsem = (r0, r1)
        esem = (e0, e1)

        pltpu.sync_copy(src_h.at[wid], src_t)
        pltpu.sync_copy(dst_h.at[wid], dst_t)
        pltpu.sync_copy(as_h, as_t)
        pltpu.sync_copy(ad_h, ad_t)

        zf = jnp.zeros((_L16,), jnp.float32)

        def zrow(r, _):
            rows0[r] = zf
            return 0
        lax.fori_loop(0, _C, zrow, 0, unroll=8)

        for k in range(_C // 16):
            ee0[pl.ds(k * 16, 16)] = zf

        base = sid * _RPT
        for t in range(_RPT // 128):
            pltpu.sync_copy(rows0.at[pl.ds(0, 128)],
                            raw_s.at[pl.ds(base + t * 128, 128)])
            if not fold_den:
                pltpu.sync_copy(ee0.at[pl.ds(0, 128)],
                                den_s.at[pl.ds(base + t * 128, 128)])
        plsc.subcore_barrier()

        pltpu.async_copy(hx_h.at[src_t.at[0]], rows0, g0)

        def j2_body(j2, _):
            for b in range(2):
                j = j2 * 2 + b
                ob = 1 - b
                rt, et = rows[b], ees[b]

                for k in range(_C // 16):
                    sidx = src_t[j, pl.ds(k * 16, 16)]
                    didx = dst_t[j, pl.ds(k * 16, 16)]
                    e = plsc.load_gather(as_t, [sidx]) + plsc.load_gather(
                        ad_t, [didx])
                    e = jnp.where(e > 0.0, e, 0.2 * e)
                    et[pl.ds(k * 16, 16)] = _exp_f32(e)

                pltpu.make_async_copy(hx_h.at[src_t.at[j]], rt, gsem[b]).wait()

                def scale_body(r, _):
                    eev = plsc.load_gather(
                        et, [jnp.full((16,), r, jnp.int32)])
                    rt[r] = rt[r] * eev
                    return 0
                lax.fori_loop(0, _C, scale_body, 0, unroll=16)

                @pl.when(j >= 1)
                def _():
                    pltpu.make_async_copy(
                        rows[ob], raw_s.at[dst_t.at[j - 1]], rsem[ob]).wait()
                    if not fold_den:
                        pltpu.make_async_copy(
                            ees[ob], den_s.at[dst_t.at[j - 1]],
                            esem[ob]).wait()

                @pl.when(j + 1 < _NCH)
                def _():
                    pltpu.async_copy(
                        hx_h.at[src_t.at[j + 1]], rows[ob], gsem[ob])

                pltpu.async_copy(rt, raw_s.at[dst_t.at[j]], rsem[b], add=True)
                if not fold_den:
                    pltpu.async_copy(
                        et, den_s.at[dst_t.at[j]], esem[b], add=True)
            return 0

        lax.fori_loop(0, _NCH // 2, j2_body, 0)
        pltpu.make_async_copy(
            rows1, raw_s.at[dst_t.at[_NCH - 1]], r1).wait()
        if not fold_den:
            pltpu.make_async_copy(
                ee1, den_s.at[dst_t.at[_NCH - 1]], e1).wait()
        plsc.subcore_barrier()

        pltpu.sync_copy(raw_s.at[pl.ds(base, _RPT)],
                        raw_h.at[cid, pl.ds(base, _RPT)])
        pltpu.sync_copy(den_s.at[pl.ds(base, _RPT)],
                        den_h.at[cid, pl.ds(base, _RPT)])

    return kern(src3, dst3, hx, as_p, ad_p)


_L16 = 16


def _dense_first(x, W1, a1s, a1d):
    def body(x_ref, w_ref, aa_ref, hx_ref, asp_ref, adp_ref):
        h = jnp.dot(x_ref[...], w_ref[...], preferred_element_type=jnp.float32)
        hx_ref[...] = jnp.zeros((_NP, _F), jnp.float32)
        hx_ref[:_N, :] = h
        pair = jnp.dot(h, aa_ref[...], preferred_element_type=jnp.float32)
        asp_ref[...] = jnp.zeros((_NP,), jnp.float32)
        adp_ref[...] = jnp.zeros((_NP,), jnp.float32)
        asp_ref[:_N] = pair[:, 0]
        adp_ref[:_N] = pair[:, 1]

    return pl.pallas_call(
        body,
        out_shape=[
            jax.ShapeDtypeStruct((_NP, _F), jnp.float32),
            jax.ShapeDtypeStruct((_NP,), jnp.float32),
            jax.ShapeDtypeStruct((_NP,), jnp.float32),
        ],
    )(x, W1, jnp.stack([a1s, a1d], axis=1))


def _dense_mid(raw, den, b, W, a_s, a_d, fin):
    fout = W.shape[1]

    def body(raw_ref, den_ref, b_ref, w_ref, aa_ref,
             hx_ref, asp_ref, adp_ref):
        rawv = raw_ref[0, :_N, :fin] + raw_ref[1, :_N, :fin]
        if fin == _F:
            denv = den_ref[0, :_N] + den_ref[1, :_N]
        else:
            denv = raw_ref[0, :_N, 8] + raw_ref[1, :_N, 8]
        z = rawv / (denv[:, None] + 1e-16) + b_ref[...]
        z = jnp.maximum(z, 0.0)
        h = jnp.dot(z, w_ref[...], preferred_element_type=jnp.float32)
        hx_ref[...] = jnp.zeros((_NP, _F), jnp.float32)
        hx_ref[:_N, :fout] = h
        hx_ref[:_N, 8] = jnp.ones((_N,), jnp.float32)
        pair = jnp.dot(h, aa_ref[...], preferred_element_type=jnp.float32)
        asp_ref[...] = jnp.zeros((_NP,), jnp.float32)
        adp_ref[...] = jnp.zeros((_NP,), jnp.float32)
        asp_ref[:_N] = pair[:, 0]
        adp_ref[:_N] = pair[:, 1]

    return pl.pallas_call(
        body,
        out_shape=[
            jax.ShapeDtypeStruct((_NP, _F), jnp.float32),
            jax.ShapeDtypeStruct((_NP,), jnp.float32),
            jax.ShapeDtypeStruct((_NP,), jnp.float32),
        ],
    )(raw, den, b, W, jnp.stack([a_s, a_d], axis=1))


def _dense_last(raw, den, b, Wfc, bfc):
    def body(raw_ref, den_ref, b_ref, w_ref, bfc_ref, o_ref):
        rawv = raw_ref[0, :_N, :8] + raw_ref[1, :_N, :8]
        denv = raw_ref[0, :_N, 8] + raw_ref[1, :_N, 8]
        z = rawv / (denv[:, None] + 1e-16) + b_ref[...]
        z = jnp.maximum(z, 0.0)
        o_ref[...] = jnp.dot(z, w_ref[...],
                             preferred_element_type=jnp.float32) + bfc_ref[...]

    return pl.pallas_call(
        body,
        out_shape=jax.ShapeDtypeStruct((_N, 1), jnp.float32),
    )(raw, den, b, Wfc, bfc[None, :])


def kernel(x, edge_index, W1, a1s, a1d, b1, W2, a2s, a2d, b2, W3, a3s, a3d, b3, Wfc, bfc):
    src = edge_index[:, 0]
    dst = edge_index[:, 1]
    padn = jnp.full((_EPAD - _E,), _N, jnp.int32)
    src3 = jnp.concatenate([src, padn]).reshape(_NW, _NCH, _C)
    dst3 = jnp.concatenate([dst, padn]).reshape(_NW, _NCH, _C)

    hx, asp, adp = _dense_first(x, W1, a1s, a1d)
    raw, den = _sc_edge_pass(src3, dst3, hx, asp, adp, False)
    hx, asp, adp = _dense_mid(raw, den, b1, W2, a2s, a2d, _F)
    raw, den = _sc_edge_pass(src3, dst3, hx, asp, adp, True)
    hx, asp, adp = _dense_mid(raw, den, b2, W3, a3s, a3d, 8)
    raw, den = _sc_edge_pass(src3, dst3, hx, asp, adp, True)
    return _dense_last(raw, den, b3, Wfc, bfc)


# speedup vs baseline: 1.2493x; 1.0072x over previous
import functools

import jax
import jax.numpy as jnp
from jax import lax
from jax.experimental import pallas as pl
from jax.experimental.pallas import tpu as pltpu
from jax.experimental.pallas import tpu_sc as plsc

_N = 10000
_E = 320000


def _exp_f32(x):
    t = x * 1.4426950408889634
    nf = (t + 12582912.0) - 12582912.0
    f = (t - nf) * 0.6931471805599453
    p = 1.0 + f * (1.0 + f * (0.5 + f * (1.0 / 6.0 + f * (
        1.0 / 24.0 + f * (1.0 / 120.0 + f * (1.0 / 720.0))))))
    n = nf.astype(jnp.int32)
    scale = jax.lax.bitcast_convert_type(
        jax.lax.shift_left(n + 127, 23), jnp.float32)
    return p * scale
_NC = 2
_NS = 16
_NW = _NC * _NS
_C = 1024
_NCH = 10
_EPT = _NCH * _C
_EPAD = _NW * _EPT
_NP = 10240
_RPT = _NP // _NS
_F = 16


def _sc_edge_pass(src3, dst3, hx, as_p, ad_p, fold_den):
    mesh = plsc.VectorSubcoreMesh(core_axis_name="c", subcore_axis_name="s")

    @functools.partial(
        pl.kernel,
        mesh=mesh,
        compiler_params=pltpu.CompilerParams(needs_layout_passes=False,
                                             use_tc_tiling_on_sc=False),
        out_type=[
            jax.ShapeDtypeStruct((_NC, _NP, _F), jnp.float32),
            jax.ShapeDtypeStruct((_NC, _NP), jnp.float32),
        ],
        scratch_types=[
            pltpu.VMEM((_NCH, _C), jnp.int32),
            pltpu.VMEM((_NCH, _C), jnp.int32),
            pltpu.VMEM((_NP,), jnp.float32),
            pltpu.VMEM((_NP,), jnp.float32),
            pltpu.VMEM((_C, _F), jnp.float32),
            pltpu.VMEM((_C, _F), jnp.float32),
            pltpu.VMEM((_C,), jnp.float32),
            pltpu.VMEM((_C,), jnp.float32),
            pltpu.VMEM_SHARED((_NP, _F), jnp.float32),
            pltpu.VMEM_SHARED((_NP,), jnp.float32),
            pltpu.SemaphoreType.DMA,
            pltpu.SemaphoreType.DMA,
            pltpu.SemaphoreType.DMA,
            pltpu.SemaphoreType.DMA,
            pltpu.SemaphoreType.DMA,
            pltpu.SemaphoreType.DMA,
        ],
    )
    def kern(src_h, dst_h, hx_h, as_h, ad_h, raw_h, den_h,
             src_t, dst_t, as_t, ad_t, rows0, rows1, ee0, ee1,
             raw_s, den_s, g0, g1, r0, r1, e0, e1):
        cid = lax.axis_index("c")
        sid = lax.axis_index("s")
        wid = sid * _NC + cid
        rows = (rows0, rows1)
        ees = (ee0, ee1)
        gsem = (g0, g1)
        rsem = (r0, r1)
        esem = (e0, e1)

        pltpu.sync_copy(src_h.at[wid], src_t)
        pltpu.sync_copy(dst_h.at[wid], dst_t)
        pltpu.sync_copy(as_h, as_t)
        pltpu.sync_copy(ad_h, ad_t)

        zf = jnp.zeros((_L16,), jnp.float32)

        def zrow(r, _):
            rows0[r] = zf
            return 0
        lax.fori_loop(0, _C, zrow, 0, unroll=8)

        for k in range(_C // 16):
            ee0[pl.ds(k * 16, 16)] = zf

        base = sid * _RPT
        for t in range(_RPT // 128):
            pltpu.sync_copy(rows0.at[pl.ds(0, 128)],
                            raw_s.at[pl.ds(base + t * 128, 128)])
            if not fold_den:
                pltpu.sync_copy(ee0.at[pl.ds(0, 128)],
                                den_s.at[pl.ds(base + t * 128, 128)])
        plsc.subcore_barrier()

        pltpu.async_copy(hx_h.at[src_t.at[0]], rows0, g0)

        def j2_body(j2, _):
            for b in range(2):
                j = j2 * 2 + b
                ob = 1 - b
                rt, et = rows[b], ees[b]

                for k in range(_C // 16):
                    sidx = src_t[j, pl.ds(k * 16, 16)]
                    didx = dst_t[j, pl.ds(k * 16, 16)]
                    e = plsc.load_gather(as_t, [sidx]) + plsc.load_gather(
                        ad_t, [didx])
                    e = jnp.where(e > 0.0, e, 0.2 * e)
                    et[pl.ds(k * 16, 16)] = _exp_f32(e)

                pltpu.make_async_copy(hx_h.at[src_t.at[j]], rt, gsem[b]).wait()

                def scale_body(r, _):
                    eev = plsc.load_gather(
                        et, [jnp.full((16,), r, jnp.int32)])
                    rt[r] = rt[r] * eev
                    return 0
                lax.fori_loop(0, _C, scale_body, 0, unroll=16)

                @pl.when(j >= 1)
                def _():
                    pltpu.make_async_copy(
                        rows[ob], raw_s.at[dst_t.at[j - 1]], rsem[ob]).wait()
                    if not fold_den:
                        pltpu.make_async_copy(
                            ees[ob], den_s.at[dst_t.at[j - 1]],
                            esem[ob]).wait()

                @pl.when(j + 1 < _NCH)
                def _():
                    pltpu.async_copy(
                        hx_h.at[src_t.at[j + 1]], rows[ob], gsem[ob])

                pltpu.async_copy(rt, raw_s.at[dst_t.at[j]], rsem[b], add=True)
                if not fold_den:
                    pltpu.async_copy(
                        et, den_s.at[dst_t.at[j]], esem[b], add=True)
            return 0

        lax.fori_loop(0, _NCH // 2, j2_body, 0)
        pltpu.make_async_copy(
            rows1, raw_s.at[dst_t.at[_NCH - 1]], r1).wait()
        if not fold_den:
            pltpu.make_async_copy(
                ee1, den_s.at[dst_t.at[_NCH - 1]], e1).wait()
        plsc.subcore_barrier()

        pltpu.sync_copy(raw_s.at[pl.ds(base, _RPT)],
                        raw_h.at[cid, pl.ds(base, _RPT)])
        pltpu.sync_copy(den_s.at[pl.ds(base, _RPT)],
                        den_h.at[cid, pl.ds(base, _RPT)])

    return kern(src3, dst3, hx, as_p, ad_p)


_L16 = 16


def _dense_first(x, W1, a1s, a1d):
    def body(x_ref, w_ref, aa_ref, hx_ref, asp_ref, adp_ref):
        h = jnp.dot(x_ref[...], w_ref[...], preferred_element_type=jnp.float32)
        hx_ref[...] = jnp.zeros((_NP, _F), jnp.float32)
        hx_ref[:_N, :] = h
        pair = jnp.dot(h, aa_ref[...], preferred_element_type=jnp.float32)
        asp_ref[...] = jnp.zeros((_NP,), jnp.float32)
        adp_ref[...] = jnp.zeros((_NP,), jnp.float32)
        asp_ref[:_N] = pair[:, 0]
        adp_ref[:_N] = pair[:, 1]

    return pl.pallas_call(
        body,
        out_shape=[
            jax.ShapeDtypeStruct((_NP, _F), jnp.float32),
            jax.ShapeDtypeStruct((_NP,), jnp.float32),
            jax.ShapeDtypeStruct((_NP,), jnp.float32),
        ],
    )(x, W1, jnp.stack([a1s, a1d], axis=1))


def _dense_mid(raw, den, b, W, a_s, a_d, fin):
    fout = W.shape[1]

    def body(raw_ref, den_ref, b_ref, w_ref, aa_ref,
             hx_ref, asp_ref, adp_ref):
        rawv = raw_ref[0, :_N, :fin] + raw_ref[1, :_N, :fin]
        if fin == _F:
            denv = den_ref[0, :_N] + den_ref[1, :_N]
        else:
            denv = raw_ref[0, :_N, 8] + raw_ref[1, :_N, 8]
        z = rawv / (denv[:, None] + 1e-16) + b_ref[...]
        z = jnp.maximum(z, 0.0)
        h = jnp.dot(z, w_ref[...], preferred_element_type=jnp.float32)
        hx_ref[...] = jnp.zeros((_NP, _F), jnp.float32)
        hx_ref[:_N, :fout] = h
        hx_ref[:_N, 8] = jnp.ones((_N,), jnp.float32)
        pair = jnp.dot(h, aa_ref[...], preferred_element_type=jnp.float32)
        asp_ref[...] = jnp.zeros((_NP,), jnp.float32)
        adp_ref[...] = jnp.zeros((_NP,), jnp.float32)
        asp_ref[:_N] = pair[:, 0]
        adp_ref[:_N] = pair[:, 1]

    return pl.pallas_call(
        body,
        out_shape=[
            jax.ShapeDtypeStruct((_NP, _F), jnp.float32),
            jax.ShapeDtypeStruct((_NP,), jnp.float32),
            jax.ShapeDtypeStruct((_NP,), jnp.float32),
        ],
    )(raw, den, b, W, jnp.stack([a_s, a_d], axis=1))


def _dense_last(raw, den, b, Wfc, bfc):
    def body(raw_ref, den_ref, b_ref, w_ref, bfc_ref, o_ref):
        rawv = raw_ref[0, :_N, :8] + raw_ref[1, :_N, :8]
        denv = raw_ref[0, :_N, 8] + raw_ref[1, :_N, 8]
        z = rawv / (denv[:, None] + 1e-16) + b_ref[...]
        z = jnp.maximum(z, 0.0)
        o_ref[...] = jnp.dot(z, w_ref[...],
                             preferred_element_type=jnp.float32) + bfc_ref[...]

    return pl.pallas_call(
        body,
        out_shape=jax.ShapeDtypeStruct((_N, 1), jnp.float32),
    )(raw, den, b, Wfc, bfc[None, :])


def kernel(x, edge_index, W1, a1s, a1d, b1, W2, a2s, a2d, b2, W3, a3s, a3d, b3, Wfc, bfc):
    src = edge_index[:, 0]
    dst = edge_index[:, 1]
    padn = jnp.full((_EPAD - _E,), _N, jnp.int32)
    src3 = jnp.concatenate([src, padn]).reshape(_NW, _NCH, _C)
    dst3 = jnp.concatenate([dst, padn]).reshape(_NW, _NCH, _C)

    hx, asp, adp = _dense_first(x, W1, a1s, a1d)
    raw, den = _sc_edge_pass(src3, dst3, hx, asp, adp, False)
    hx, asp, adp = _dense_mid(raw, den, b1, W2, a2s, a2d, _F)
    raw, den = _sc_edge_pass(src3, dst3, hx, asp, adp, True)
    hx, asp, adp = _dense_mid(raw, den, b2, W3, a3s, a3d, 8)
    raw, den = _sc_edge_pass(src3, dst3, hx, asp, adp, True)
    return _dense_last(raw, den, b3, Wfc, bfc)
